# baseline (speedup 1.0000x reference)
import functools

import jax
import jax.numpy as jnp
from jax import lax
from jax.experimental import pallas as pl
from jax.experimental.pallas import tpu as pltpu
from jax.experimental.pallas import tpu_sc as plsc

N = 10000
E = 320000
B = 16
S = 128
NE = 4
STEPS = 2

NC = 2
NS = 16
NW = NC * NS
EPW = E // NW
K = 80
ITERS = EPW // K
NPAD = 10240
RPS = NPAD // NS

R = 1000
NB = N // R
NP = 10240
RP = 1024
NBP = NP // RP

_f32 = jnp.float32


def _sc_pre_body(ef_hbm, ed3_hbm, nodes_hbm, lin_hbm, zns_hbm, ones_hbm,
                 ef_out, deg_out, nn_out,
                 acc, didx2d, rows_a, rows_b, lin_v, nn_v,
                 sem_a, sem_b, sem_sa, sem_sb):
    c = lax.axis_index("c")
    s = lax.axis_index("s")
    wid = c * NS + s
    r0 = s * RPS

    @pl.when(jnp.logical_and(c == 0, s == 0))
    def _():
        pltpu.sync_copy(lin_hbm, lin_v)
        pltpu.async_copy(nodes_hbm.at[lin_v], nn_v, sem_a).wait()
        pltpu.sync_copy(nn_v, nn_out)

    pltpu.sync_copy(ed3_hbm.at[wid], didx2d)
    pltpu.sync_copy(zns_hbm.at[pl.ds(r0, RPS)], acc.at[pl.ds(r0, RPS)])
    plsc.subcore_barrier()

    def src(i):
        return ef_hbm.at[pl.ds(wid * EPW + i * K, K)]

    pltpu.async_copy(src(0), rows_a, sem_a)
    pltpu.async_copy(src(1), rows_b, sem_b)

    def body(j, carry):
        i0 = 2 * j
        pltpu.make_async_copy(src(i0), rows_a, sem_a).wait()
        pltpu.async_copy(rows_a, acc.at[didx2d.at[i0]], sem_sa, add=True)
        pltpu.make_async_copy(src(i0 + 1), rows_b, sem_b).wait()
        pltpu.async_copy(rows_b, acc.at[didx2d.at[i0 + 1]], sem_sb, add=True)
        pltpu.make_async_copy(rows_a, acc.at[didx2d.at[i0]], sem_sa).wait()

        @pl.when(i0 + 2 < ITERS)
        def _():
            pltpu.async_copy(src(i0 + 2), rows_a, sem_a)
        pltpu.make_async_copy(rows_b, acc.at[didx2d.at[i0 + 1]], sem_sb).wait()

        @pl.when(i0 + 3 < ITERS)
        def _():
            pltpu.async_copy(src(i0 + 3), rows_b, sem_b)
        return carry

    lax.fori_loop(0, (ITERS - 1) // 2, body, 0)
    pltpu.make_async_copy(src(ITERS - 1), rows_a, sem_a).wait()
    pltpu.sync_copy(rows_a, acc.at[didx2d.at[ITERS - 1]], add=True)
    plsc.subcore_barrier()
    pltpu.sync_copy(acc.at[pl.ds(r0, RPS)], ef_out.at[pl.ds(c * NPAD + r0, RPS)])
    plsc.subcore_barrier()

    pltpu.sync_copy(zns_hbm.at[pl.ds(r0, RPS)], acc.at[pl.ds(r0, RPS)])
    pltpu.sync_copy(ones_hbm, rows_a)
    plsc.subcore_barrier()

    pltpu.async_copy(rows_a, acc.at[didx2d.at[0]], sem_sa, add=True)

    def body2(j, carry):
        i0 = 2 * j
        pltpu.async_copy(rows_a, acc.at[didx2d.at[i0 + 1]], sem_sb, add=True)
        pltpu.make_async_copy(rows_a, acc.at[didx2d.at[i0]], sem_sa).wait()
        pltpu.async_copy(rows_a, acc.at[didx2d.at[i0 + 2]], sem_sa, add=True)
        pltpu.make_async_copy(rows_a, acc.at[didx2d.at[i0 + 1]], sem_sb).wait()
        return carry

    lax.fori_loop(0, (ITERS - 1) // 2, body2, 0)
    pltpu.make_async_copy(rows_a, acc.at[didx2d.at[ITERS - 1]], sem_sa).wait()
    plsc.subcore_barrier()
    pltpu.sync_copy(acc.at[pl.ds(r0, RPS)], deg_out.at[pl.ds(c * NPAD + r0, RPS)])


def _sc_gather_body(nodes_hbm, es_hbm, ed3_hbm, zns_hbm, g_out,
                    acc, sidx_all, didx2d, rows_a, rows_b,
                    sem_a, sem_b, sem_sa, sem_sb):
    c = lax.axis_index("c")
    s = lax.axis_index("s")
    wid = c * NS + s
    r0 = s * RPS
    pltpu.sync_copy(es_hbm.at[pl.ds(wid * EPW, EPW)], sidx_all)
    pltpu.sync_copy(ed3_hbm.at[wid], didx2d)
    pltpu.sync_copy(zns_hbm.at[pl.ds(r0, RPS)], acc.at[pl.ds(r0, RPS)])
    plsc.subcore_barrier()

    def src(i):
        return nodes_hbm.at[sidx_all.at[pl.ds(i * K, K)]]

    pltpu.async_copy(src(0), rows_a, sem_a)
    pltpu.async_copy(src(1), rows_b, sem_b)

    def body(j, carry):
        i0 = 2 * j
        pltpu.make_async_copy(src(i0), rows_a, sem_a).wait()
        pltpu.async_copy(rows_a, acc.at[didx2d.at[i0]], sem_sa, add=True)
        pltpu.make_async_copy(src(i0 + 1), rows_b, sem_b).wait()
        pltpu.async_copy(rows_b, acc.at[didx2d.at[i0 + 1]], sem_sb, add=True)
        pltpu.make_async_copy(rows_a, acc.at[didx2d.at[i0]], sem_sa).wait()

        @pl.when(i0 + 2 < ITERS)
        def _():
            pltpu.async_copy(src(i0 + 2), rows_a, sem_a)
        pltpu.make_async_copy(rows_b, acc.at[didx2d.at[i0 + 1]], sem_sb).wait()

        @pl.when(i0 + 3 < ITERS)
        def _():
            pltpu.async_copy(src(i0 + 3), rows_b, sem_b)
        return carry

    lax.fori_loop(0, (ITERS - 1) // 2, body, 0)
    pltpu.make_async_copy(src(ITERS - 1), rows_a, sem_a).wait()
    pltpu.sync_copy(rows_a, acc.at[didx2d.at[ITERS - 1]], add=True)
    plsc.subcore_barrier()
    pltpu.sync_copy(acc.at[pl.ds(r0, RPS)], g_out.at[pl.ds(c * NPAD + r0, RPS)])


@functools.cache
def _build_sc_kernels():
    mesh = plsc.VectorSubcoreMesh(core_axis_name="c", subcore_axis_name="s")
    sc_pre = pl.kernel(
        _sc_pre_body,
        out_type=(
            jax.ShapeDtypeStruct((NC * NPAD, S), _f32),
            jax.ShapeDtypeStruct((NC * NPAD, S), _f32),
            jax.ShapeDtypeStruct((B, S), _f32),
        ),
        mesh=mesh,
        scratch_types=[
            pltpu.VMEM_SHARED((NPAD, S), _f32),
            pltpu.VMEM((ITERS, K), jnp.int32),
            pltpu.VMEM((K, S), _f32),
            pltpu.VMEM((K, S), _f32),
            pltpu.VMEM((B,), jnp.int32),
            pltpu.VMEM((B, S), _f32),
            pltpu.SemaphoreType.DMA,
            pltpu.SemaphoreType.DMA,
            pltpu.SemaphoreType.DMA,
            pltpu.SemaphoreType.DMA,
        ],
    )
    sc_gather = pl.kernel(
        _sc_gather_body,
        out_type=jax.ShapeDtypeStruct((NC * NPAD, S), _f32),
        mesh=mesh,
        scratch_types=[
            pltpu.VMEM_SHARED((NPAD, S), _f32),
            pltpu.VMEM((EPW,), jnp.int32),
            pltpu.VMEM((ITERS, K), jnp.int32),
            pltpu.VMEM((K, S), _f32),
            pltpu.VMEM((K, S), _f32),
            pltpu.SemaphoreType.DMA,
            pltpu.SemaphoreType.DMA,
            pltpu.SemaphoreType.DMA,
            pltpu.SemaphoreType.DMA,
        ],
    )
    return sc_pre, sc_gather


def _sc_pre(*args):
    return _build_sc_kernels()[0](*args)


def _sc_gather(*args):
    return _build_sc_kernels()[1](*args)


def _step_body(nodes_ref, gp_ref, efp_ref, dp_ref, omt_ref, runf_ref,
               wsrc_ref, wdst_ref, bdst_ref, wef_ref,
               wih_ref, bih_ref, whh_ref, bhh_ref, out_ref):
    h = nodes_ref[...]
    g = gp_ref[0] + gp_ref[1]
    efa = efp_ref[0] + efp_ref[1]
    deg = dp_ref[0, :, :1] + dp_ref[1, :, :1]
    dst_t = jnp.dot(h, wdst_ref[...], preferred_element_type=_f32) + bdst_ref[...]
    inputs = (jnp.dot(g, wsrc_ref[...], preferred_element_type=_f32)
              + deg * dst_t
              + jnp.dot(efa, wef_ref[...], preferred_element_type=_f32))
    gi = jnp.dot(inputs, wih_ref[...], preferred_element_type=_f32) + bih_ref[...]
    gh = jnp.dot(h, whh_ref[...], preferred_element_type=_f32) + bhh_ref[...]
    r = jax.nn.sigmoid(gi[:, :S] + gh[:, :S])
    z = jax.nn.sigmoid(gi[:, S:2 * S] + gh[:, S:2 * S])
    n = jnp.tanh(gi[:, 2 * S:] + r * gh[:, 2 * S:])
    new = (1.0 - z) * n + z * h
    nm = jnp.sum(omt_ref[...].astype(_f32) * runf_ref[...],
                 axis=1, keepdims=True) > 0.0
    out_ref[...] = jnp.where(nm, new, h)


_step_call = pl.pallas_call(
    _step_body,
    grid=(NB,),
    in_specs=[
        pl.BlockSpec((R, S), lambda i: (i, 0)),
        pl.BlockSpec((NC, R, S), lambda i: (0, i, 0)),
        pl.BlockSpec((NC, R, S), lambda i: (0, i, 0)),
        pl.BlockSpec((NC, R, S), lambda i: (0, i, 0)),
        pl.BlockSpec((R, B), lambda i: (i, 0)),
        pl.BlockSpec((1, B), lambda i: (0, 0)),
        pl.BlockSpec((S, 2 * S), lambda i: (0, 0)),
        pl.BlockSpec((S, 2 * S), lambda i: (0, 0)),
        pl.BlockSpec((1, 2 * S), lambda i: (0, 0)),
        pl.BlockSpec((S, 2 * S), lambda i: (0, 0)),
        pl.BlockSpec((2 * S, 3 * S), lambda i: (0, 0)),
        pl.BlockSpec((1, 3 * S), lambda i: (0, 0)),
        pl.BlockSpec((S, 3 * S), lambda i: (0, 0)),
        pl.BlockSpec((1, 3 * S), lambda i: (0, 0)),
    ],
    out_specs=pl.BlockSpec((R, S), lambda i: (i, 0)),
    out_shape=jax.ShapeDtypeStruct((N, S), _f32),
)


def _head_body(x_ref, om_ref, nn_ref, wg_ref, bg_ref, wa_ref, ba_ref,
               wt_ref, bt_ref, wn_ref, wae_ref, bae_ref, wan_ref,
               ml_ref, agg_ref, ne_ref):
    i = pl.program_id(0)
    x = x_ref[...]
    gates = jax.nn.sigmoid(
        jnp.dot(x, wg_ref[...], preferred_element_type=_f32) + bg_ref[...])
    data = jnp.dot(x, wa_ref[...], preferred_element_type=_f32) + ba_ref[...]
    dm = data * gates
    om = om_ref[...]

    @pl.when(i == 0)
    def _():
        agg_ref[...] = jnp.zeros_like(agg_ref)

    agg_ref[...] += jnp.dot(om.astype(_f32), dm, preferred_element_type=_f32)

    lt = lax.dot_general(wt_ref[...], x, (((1,), (1,)), ((), ())),
                         preferred_element_type=_f32)
    nn8 = jnp.dot(nn_ref[...], wn_ref[...], preferred_element_type=_f32)
    neg = jnp.float32(-1e9)
    for t in range(NE):
        val = lt[t:t + 1, :] + bt_ref[:, t:t + 1] + nn8[:, t:t + 1]
        ml_ref[t] = jnp.where(om > 0, val, neg)

    @pl.when(i == NBP - 1)
    def _():
        ne_ref[...] = (
            jnp.dot(agg_ref[...], wae_ref[...], preferred_element_type=_f32)
            + jnp.dot(nn_ref[...], wan_ref[...], preferred_element_type=_f32)
            + bae_ref[...])


_head_call = pl.pallas_call(
    _head_body,
    grid=(NBP,),
    in_specs=[
        pl.BlockSpec((RP, S), lambda i: (i, 0)),
        pl.BlockSpec((B, RP), lambda i: (0, i)),
        pl.BlockSpec((B, S), lambda i: (0, 0)),
        pl.BlockSpec((S, S), lambda i: (0, 0)),
        pl.BlockSpec((1, S), lambda i: (0, 0)),
        pl.BlockSpec((S, S), lambda i: (0, 0)),
        pl.BlockSpec((1, S), lambda i: (0, 0)),
        pl.BlockSpec((8, S), lambda i: (0, 0)),
        pl.BlockSpec((1, 8), lambda i: (0, 0)),
        pl.BlockSpec((S, 8), lambda i: (0, 0)),
        pl.BlockSpec((S, 8), lambda i: (0, 0)),
        pl.BlockSpec((1, 8), lambda i: (0, 0)),
        pl.BlockSpec((S, 8), lambda i: (0, 0)),
    ],
    out_specs=[
        pl.BlockSpec((NE, B, RP), lambda i: (0, 0, i)),
        pl.BlockSpec((B, S), lambda i: (0, 0)),
        pl.BlockSpec((B, 8), lambda i: (0, 0)),
    ],
    out_shape=[
        jax.ShapeDtypeStruct((NE, B, NP), _f32),
        jax.ShapeDtypeStruct((B, S), _f32),
        jax.ShapeDtypeStruct((B, 8), _f32),
    ],
)


def kernel(nodes, edge_features, edge_source, edge_dest, owner_masks,
           last_inserted_node, running, params):
    p = params
    es = edge_source.astype(jnp.int32)
    ed3 = edge_dest.astype(jnp.int32).reshape(NW, ITERS, K)
    lin = last_inserted_node.astype(jnp.int32)
    zns = jnp.zeros((NPAD, S), _f32)
    ones_k = jnp.ones((K, S), _f32)

    ef_part, deg_part, new_nodes = _sc_pre(
        edge_features, ed3, nodes, lin, zns, ones_k)
    ef_part = ef_part.reshape(NC, NPAD, S)
    deg_part = deg_part.reshape(NC, NPAD, S)

    omt = owner_masks.T
    runf = running.astype(_f32).reshape(1, B)

    h = nodes
    for s in range(STEPS):
        g_part = _sc_gather(h, es, ed3, zns).reshape(NC, NPAD, S)
        h = _step_call(
            h, g_part, ef_part, deg_part, omt, runf,
            p[f"W_src{s}"].T, p[f"W_dst{s}"].T, p[f"b_dst{s}"].reshape(1, 2 * S),
            p[f"W_ef{s}"].T, p[f"W_ih{s}"].T, p[f"b_ih{s}"].reshape(1, 3 * S),
            p[f"W_hh{s}"].T, p[f"b_hh{s}"].reshape(1, 3 * S))

    x_p = jnp.concatenate([h, jnp.zeros((NP - N, S), _f32)], axis=0)
    om_p = jnp.concatenate(
        [owner_masks, jnp.zeros((B, NP - N), owner_masks.dtype)], axis=1)

    wt8 = jnp.concatenate([p["W_t"], jnp.zeros((8 - NE, S), _f32)], axis=0)
    bt8 = jnp.concatenate([p["b_t"], jnp.zeros((8 - NE,), _f32)]).reshape(1, 8)
    wn8 = jnp.concatenate([p["W_n"].T, jnp.zeros((S, 8 - NE), _f32)], axis=1)
    wae8 = jnp.concatenate([p["W_ae"].T, jnp.zeros((S, 7), _f32)], axis=1)
    bae8 = jnp.concatenate([p["b_ae"], jnp.zeros((7,), _f32)]).reshape(1, 8)
    wan8 = jnp.concatenate([p["W_an"].T, jnp.zeros((S, 7), _f32)], axis=1)

    ml4, _agg, ne8 = _head_call(
        x_p, om_p, new_nodes,
        p["W_agg_g"].T, p["b_agg_g"].reshape(1, S),
        p["W_agg_t"].T, p["b_agg_t"].reshape(1, S),
        wt8, bt8, wn8, wae8, bae8, wan8)

    masked_logits = ml4[:, :, :N].transpose(1, 2, 0).reshape(B, N * NE)
    new_edge_needed = ne8[:, 0]
    return new_edge_needed, masked_logits


# speedup vs baseline: 1.2375x; 1.2375x over previous
import functools

import jax
import jax.numpy as jnp
from jax import lax
from jax.experimental import pallas as pl
from jax.experimental.pallas import tpu as pltpu
from jax.experimental.pallas import tpu_sc as plsc

N = 10000
E = 320000
B = 16
S = 128
NE = 4
STEPS = 2

NC = 2
NS = 16
NW = NC * NS
EPW = E // NW
K = 80
ITERS = EPW // K
NPAD = 10240
RPS = NPAD // NS

R = 1000
NB = N // R
NP = 10240
RP = 1024
NBP = NP // RP

_f32 = jnp.float32


def _sc_pre_body(ef_hbm, ed3_hbm, nodes_hbm, lin_hbm, zns_hbm, ones_hbm, es_hbm,
                 ef_out, deg_out, g0_out, nn_out,
                 acc, didx2d, sidx_all, rows_a, rows_b, lin_v, nn_v,
                 sem_a, sem_b):
    c = lax.axis_index("c")
    s = lax.axis_index("s")
    wid = c * NS + s
    r0 = s * RPS

    @pl.when(jnp.logical_and(c == 0, s == 0))
    def _():
        pltpu.sync_copy(lin_hbm, lin_v)
        pltpu.async_copy(nodes_hbm.at[lin_v], nn_v, sem_a).wait()
        pltpu.sync_copy(nn_v, nn_out)

    pltpu.sync_copy(ed3_hbm.at[wid], didx2d)
    pltpu.sync_copy(es_hbm.at[pl.ds(wid * EPW, EPW)], sidx_all)
    pltpu.sync_copy(zns_hbm.at[pl.ds(r0, RPS)], acc.at[pl.ds(r0, RPS)])
    plsc.subcore_barrier()

    def src(i):
        return ef_hbm.at[pl.ds(wid * EPW + i * K, K)]

    pltpu.async_copy(src(0), rows_a, sem_a)

    def body(j, carry):
        i0 = 2 * j
        pltpu.async_copy(src(i0 + 1), rows_b, sem_b)
        pltpu.make_async_copy(src(i0), rows_a, sem_a).wait()
        pltpu.sync_copy(rows_a, acc.at[didx2d.at[i0]], add=True)
        pltpu.async_copy(src(i0 + 2), rows_a, sem_a)
        pltpu.make_async_copy(src(i0 + 1), rows_b, sem_b).wait()
        pltpu.sync_copy(rows_b, acc.at[didx2d.at[i0 + 1]], add=True)
        return carry

    lax.fori_loop(0, (ITERS - 1) // 2, body, 0)
    pltpu.make_async_copy(src(ITERS - 1), rows_a, sem_a).wait()
    pltpu.sync_copy(rows_a, acc.at[didx2d.at[ITERS - 1]], add=True)
    plsc.subcore_barrier()
    pltpu.sync_copy(acc.at[pl.ds(r0, RPS)], ef_out.at[pl.ds(c * NPAD + r0, RPS)])
    plsc.subcore_barrier()

    pltpu.sync_copy(zns_hbm.at[pl.ds(r0, RPS)], acc.at[pl.ds(r0, RPS)])
    pltpu.sync_copy(ones_hbm, rows_a)
    plsc.subcore_barrier()

    def body2(i, carry):
        pltpu.sync_copy(rows_a, acc.at[didx2d.at[i]], add=True)
        return carry

    lax.fori_loop(0, ITERS, body2, 0)
    plsc.subcore_barrier()
    pltpu.sync_copy(acc.at[pl.ds(r0, RPS)], deg_out.at[pl.ds(c * NPAD + r0, RPS)])
    plsc.subcore_barrier()

    pltpu.sync_copy(zns_hbm.at[pl.ds(r0, RPS)], acc.at[pl.ds(r0, RPS)])
    plsc.subcore_barrier()

    def gsrc(i):
        return nodes_hbm.at[sidx_all.at[pl.ds(i * K, K)]]

    pltpu.async_copy(gsrc(0), rows_a, sem_a)

    def body3(j, carry):
        i0 = 2 * j
        pltpu.async_copy(gsrc(i0 + 1), rows_b, sem_b)
        pltpu.make_async_copy(gsrc(i0), rows_a, sem_a).wait()
        pltpu.sync_copy(rows_a, acc.at[didx2d.at[i0]], add=True)
        pltpu.async_copy(gsrc(i0 + 2), rows_a, sem_a)
        pltpu.make_async_copy(gsrc(i0 + 1), rows_b, sem_b).wait()
        pltpu.sync_copy(rows_b, acc.at[didx2d.at[i0 + 1]], add=True)
        return carry

    lax.fori_loop(0, (ITERS - 1) // 2, body3, 0)
    pltpu.make_async_copy(gsrc(ITERS - 1), rows_a, sem_a).wait()
    pltpu.sync_copy(rows_a, acc.at[didx2d.at[ITERS - 1]], add=True)
    plsc.subcore_barrier()
    pltpu.sync_copy(acc.at[pl.ds(r0, RPS)], g0_out.at[pl.ds(c * NPAD + r0, RPS)])


def _sc_gather_body(nodes_hbm, es_hbm, ed3_hbm, zns_hbm, g_out,
                    acc, sidx_all, didx2d, rows_a, rows_b,
                    sem_a, sem_b):
    c = lax.axis_index("c")
    s = lax.axis_index("s")
    wid = c * NS + s
    r0 = s * RPS
    pltpu.sync_copy(es_hbm.at[pl.ds(wid * EPW, EPW)], sidx_all)
    pltpu.sync_copy(ed3_hbm.at[wid], didx2d)
    pltpu.sync_copy(zns_hbm.at[pl.ds(r0, RPS)], acc.at[pl.ds(r0, RPS)])
    plsc.subcore_barrier()

    def src(i):
        return nodes_hbm.at[sidx_all.at[pl.ds(i * K, K)]]

    pltpu.async_copy(src(0), rows_a, sem_a)

    def body(j, carry):
        i0 = 2 * j
        pltpu.async_copy(src(i0 + 1), rows_b, sem_b)
        pltpu.make_async_copy(src(i0), rows_a, sem_a).wait()
        pltpu.sync_copy(rows_a, acc.at[didx2d.at[i0]], add=True)
        pltpu.async_copy(src(i0 + 2), rows_a, sem_a)
        pltpu.make_async_copy(src(i0 + 1), rows_b, sem_b).wait()
        pltpu.sync_copy(rows_b, acc.at[didx2d.at[i0 + 1]], add=True)
        return carry

    lax.fori_loop(0, (ITERS - 1) // 2, body, 0)
    pltpu.make_async_copy(src(ITERS - 1), rows_a, sem_a).wait()
    pltpu.sync_copy(rows_a, acc.at[didx2d.at[ITERS - 1]], add=True)
    plsc.subcore_barrier()
    pltpu.sync_copy(acc.at[pl.ds(r0, RPS)], g_out.at[pl.ds(c * NPAD + r0, RPS)])


@functools.cache
def _build_sc_kernels():
    mesh = plsc.VectorSubcoreMesh(core_axis_name="c", subcore_axis_name="s")
    sc_pre = pl.kernel(
        _sc_pre_body,
        out_type=(
            jax.ShapeDtypeStruct((NC * NPAD, S), _f32),
            jax.ShapeDtypeStruct((NC * NPAD, S), _f32),
            jax.ShapeDtypeStruct((NC * NPAD, S), _f32),
            jax.ShapeDtypeStruct((B, S), _f32),
        ),
        mesh=mesh,
        scratch_types=[
            pltpu.VMEM_SHARED((NPAD, S), _f32),
            pltpu.VMEM((ITERS, K), jnp.int32),
            pltpu.VMEM((EPW,), jnp.int32),
            pltpu.VMEM((K, S), _f32),
            pltpu.VMEM((K, S), _f32),
            pltpu.VMEM((B,), jnp.int32),
            pltpu.VMEM((B, S), _f32),
            pltpu.SemaphoreType.DMA,
            pltpu.SemaphoreType.DMA,
        ],
    )
    sc_gather = pl.kernel(
        _sc_gather_body,
        out_type=jax.ShapeDtypeStruct((NC * NPAD, S), _f32),
        mesh=mesh,
        scratch_types=[
            pltpu.VMEM_SHARED((NPAD, S), _f32),
            pltpu.VMEM((EPW,), jnp.int32),
            pltpu.VMEM((ITERS, K), jnp.int32),
            pltpu.VMEM((K, S), _f32),
            pltpu.VMEM((K, S), _f32),
            pltpu.SemaphoreType.DMA,
            pltpu.SemaphoreType.DMA,
        ],
    )
    return sc_pre, sc_gather


def _sc_pre(*args):
    return _build_sc_kernels()[0](*args)


def _sc_gather(*args):
    return _build_sc_kernels()[1](*args)


def _gru_block(nodes_ref, gp_ref, efp_ref, dp_ref, omt_ref, runf_ref,
               wsrc_ref, wdst_ref, bdst_ref, wef_ref,
               wih_ref, bih_ref, whh_ref, bhh_ref):
    h = nodes_ref[...]
    g = gp_ref[0] + gp_ref[1]
    efa = efp_ref[0] + efp_ref[1]
    deg = dp_ref[0, :, :1] + dp_ref[1, :, :1]
    dst_t = jnp.dot(h, wdst_ref[...], preferred_element_type=_f32) + bdst_ref[...]
    inputs = (jnp.dot(g, wsrc_ref[...], preferred_element_type=_f32)
              + deg * dst_t
              + jnp.dot(efa, wef_ref[...], preferred_element_type=_f32))
    gi = jnp.dot(inputs, wih_ref[...], preferred_element_type=_f32) + bih_ref[...]
    gh = jnp.dot(h, whh_ref[...], preferred_element_type=_f32) + bhh_ref[...]
    r = jax.nn.sigmoid(gi[:, :S] + gh[:, :S])
    z = jax.nn.sigmoid(gi[:, S:2 * S] + gh[:, S:2 * S])
    n = jnp.tanh(gi[:, 2 * S:] + r * gh[:, 2 * S:])
    new = (1.0 - z) * n + z * h
    nm = jnp.sum(omt_ref[...].astype(_f32) * runf_ref[...],
                 axis=1, keepdims=True) > 0.0
    return jnp.where(nm, new, h)


def _step_body(nodes_ref, gp_ref, efp_ref, dp_ref, omt_ref, runf_ref,
               wsrc_ref, wdst_ref, bdst_ref, wef_ref,
               wih_ref, bih_ref, whh_ref, bhh_ref, out_ref):
    out_ref[...] = _gru_block(nodes_ref, gp_ref, efp_ref, dp_ref, omt_ref,
                              runf_ref, wsrc_ref, wdst_ref, bdst_ref, wef_ref,
                              wih_ref, bih_ref, whh_ref, bhh_ref)


def _head_block(x, i, om_ref, nn_ref, wg_ref, bg_ref, wa_ref, ba_ref,
                wt_ref, bt_ref, wn_ref, wae_ref, bae_ref, wan_ref,
                ml_ref, agg_ref, ne_ref):
    gates = jax.nn.sigmoid(
        jnp.dot(x, wg_ref[...], preferred_element_type=_f32) + bg_ref[...])
    data = jnp.dot(x, wa_ref[...], preferred_element_type=_f32) + ba_ref[...]
    dm = data * gates
    om = om_ref[...]

    @pl.when(i == 0)
    def _():
        agg_ref[...] = jnp.zeros_like(agg_ref)

    agg_ref[...] += jnp.dot(om.astype(_f32), dm, preferred_element_type=_f32)

    lt = lax.dot_general(wt_ref[...], x, (((1,), (1,)), ((), ())),
                         preferred_element_type=_f32)
    nn8 = jnp.dot(nn_ref[...], wn_ref[...], preferred_element_type=_f32)
    neg = jnp.float32(-1e9)
    for t in range(NE):
        val = lt[t:t + 1, :] + bt_ref[:, t:t + 1] + nn8[:, t:t + 1]
        ml_ref[t] = jnp.where(om > 0, val, neg)

    @pl.when(i == NBP - 1)
    def _():
        ne_ref[...] = (
            jnp.dot(agg_ref[...], wae_ref[...], preferred_element_type=_f32)
            + jnp.dot(nn_ref[...], wan_ref[...], preferred_element_type=_f32)
            + bae_ref[...])


def _step_head_body(nodes_ref, gp_ref, efp_ref, dp_ref, omt_ref, runf_ref,
                    wsrc_ref, wdst_ref, bdst_ref, wef_ref,
                    wih_ref, bih_ref, whh_ref, bhh_ref,
                    om_ref, nn_ref, wg_ref, bg_ref, wa_ref, ba_ref,
                    wt_ref, bt_ref, wn_ref, wae_ref, bae_ref, wan_ref,
                    ml_ref, agg_ref, ne_ref):
    x = _gru_block(nodes_ref, gp_ref, efp_ref, dp_ref, omt_ref, runf_ref,
                   wsrc_ref, wdst_ref, bdst_ref, wef_ref,
                   wih_ref, bih_ref, whh_ref, bhh_ref)
    _head_block(x, pl.program_id(0), om_ref, nn_ref, wg_ref, bg_ref, wa_ref,
                ba_ref, wt_ref, bt_ref, wn_ref, wae_ref, bae_ref, wan_ref,
                ml_ref, agg_ref, ne_ref)


_step_specs = [
    pl.BlockSpec((RP, S), lambda i: (i, 0)),
    pl.BlockSpec((NC, RP, S), lambda i: (0, i, 0)),
    pl.BlockSpec((NC, RP, S), lambda i: (0, i, 0)),
    pl.BlockSpec((NC, RP, S), lambda i: (0, i, 0)),
    pl.BlockSpec((RP, B), lambda i: (i, 0)),
    pl.BlockSpec((1, B), lambda i: (0, 0)),
    pl.BlockSpec((S, 2 * S), lambda i: (0, 0)),
    pl.BlockSpec((S, 2 * S), lambda i: (0, 0)),
    pl.BlockSpec((1, 2 * S), lambda i: (0, 0)),
    pl.BlockSpec((S, 2 * S), lambda i: (0, 0)),
    pl.BlockSpec((2 * S, 3 * S), lambda i: (0, 0)),
    pl.BlockSpec((1, 3 * S), lambda i: (0, 0)),
    pl.BlockSpec((S, 3 * S), lambda i: (0, 0)),
    pl.BlockSpec((1, 3 * S), lambda i: (0, 0)),
]

_head_specs = [
    pl.BlockSpec((B, RP), lambda i: (0, i)),
    pl.BlockSpec((B, S), lambda i: (0, 0)),
    pl.BlockSpec((S, S), lambda i: (0, 0)),
    pl.BlockSpec((1, S), lambda i: (0, 0)),
    pl.BlockSpec((S, S), lambda i: (0, 0)),
    pl.BlockSpec((1, S), lambda i: (0, 0)),
    pl.BlockSpec((8, S), lambda i: (0, 0)),
    pl.BlockSpec((1, 8), lambda i: (0, 0)),
    pl.BlockSpec((S, 8), lambda i: (0, 0)),
    pl.BlockSpec((S, 8), lambda i: (0, 0)),
    pl.BlockSpec((1, 8), lambda i: (0, 0)),
    pl.BlockSpec((S, 8), lambda i: (0, 0)),
]

_step_call = pl.pallas_call(
    _step_body,
    grid=(NBP,),
    in_specs=_step_specs,
    out_specs=pl.BlockSpec((RP, S), lambda i: (i, 0)),
    out_shape=jax.ShapeDtypeStruct((NPAD, S), _f32),
)

_step_head_call = pl.pallas_call(
    _step_head_body,
    grid=(NBP,),
    in_specs=_step_specs + _head_specs,
    out_specs=[
        pl.BlockSpec((NE, B, RP), lambda i: (0, 0, i)),
        pl.BlockSpec((B, S), lambda i: (0, 0)),
        pl.BlockSpec((B, 8), lambda i: (0, 0)),
    ],
    out_shape=[
        jax.ShapeDtypeStruct((NE, B, NPAD), _f32),
        jax.ShapeDtypeStruct((B, S), _f32),
        jax.ShapeDtypeStruct((B, 8), _f32),
    ],
)


def kernel(nodes, edge_features, edge_source, edge_dest, owner_masks,
           last_inserted_node, running, params):
    p = params
    es = edge_source.astype(jnp.int32)
    ed3 = edge_dest.astype(jnp.int32).reshape(NW, ITERS, K)
    lin = last_inserted_node.astype(jnp.int32)
    zns = jnp.zeros((NPAD, S), _f32)
    ones_k = jnp.ones((K, S), _f32)

    ef_part, deg_part, g0_part, new_nodes = _sc_pre(
        edge_features, ed3, nodes, lin, zns, ones_k, es)
    ef_part = ef_part.reshape(NC, NPAD, S)
    deg_part = deg_part.reshape(NC, NPAD, S)

    om_p = jnp.concatenate(
        [owner_masks, jnp.zeros((B, NPAD - N), owner_masks.dtype)], axis=1)
    omt_p = om_p.T
    runf = running.astype(_f32).reshape(1, B)
    h = jnp.concatenate([nodes, jnp.zeros((NPAD - N, S), _f32)], axis=0)

    def step_w(s):
        return (p[f"W_src{s}"].T, p[f"W_dst{s}"].T,
                p[f"b_dst{s}"].reshape(1, 2 * S), p[f"W_ef{s}"].T,
                p[f"W_ih{s}"].T, p[f"b_ih{s}"].reshape(1, 3 * S),
                p[f"W_hh{s}"].T, p[f"b_hh{s}"].reshape(1, 3 * S))

    h = _step_call(h, g0_part.reshape(NC, NPAD, S), ef_part, deg_part,
                   omt_p, runf, *step_w(0))

    g1_part = _sc_gather(h, es, ed3, zns).reshape(NC, NPAD, S)

    wt8 = jnp.concatenate([p["W_t"], jnp.zeros((8 - NE, S), _f32)], axis=0)
    bt8 = jnp.concatenate([p["b_t"], jnp.zeros((8 - NE,), _f32)]).reshape(1, 8)
    wn8 = jnp.concatenate([p["W_n"].T, jnp.zeros((S, 8 - NE), _f32)], axis=1)
    wae8 = jnp.concatenate([p["W_ae"].T, jnp.zeros((S, 7), _f32)], axis=1)
    bae8 = jnp.concatenate([p["b_ae"], jnp.zeros((7,), _f32)]).reshape(1, 8)
    wan8 = jnp.concatenate([p["W_an"].T, jnp.zeros((S, 7), _f32)], axis=1)

    ml4, _agg, ne8 = _step_head_call(
        h, g1_part, ef_part, deg_part, omt_p, runf, *step_w(1),
        om_p, new_nodes,
        p["W_agg_g"].T, p["b_agg_g"].reshape(1, S),
        p["W_agg_t"].T, p["b_agg_t"].reshape(1, S),
        wt8, bt8, wn8, wae8, bae8, wan8)

    masked_logits = ml4[:, :, :N].transpose(1, 2, 0).reshape(B, N * NE)
    new_edge_needed = ne8[:, 0]
    return new_edge_needed, masked_logits
